# baseline (speedup 1.0000x reference)
import functools

import jax
import jax.numpy as jnp
from jax import lax
from jax.experimental import pallas as pl
from jax.experimental.pallas import tpu as pltpu
from jax.experimental.pallas import tpu_sc as plsc

_NUM_LAYERS = 3

_info = plsc.get_sparse_core_info()
_NC, _NS = _info.num_cores, _info.num_subcores
_NW = _NC * _NS


def _gather2_add(table1, idx1, table2, idx2, chunk):
    total = idx1.shape[0]
    d = table1.shape[1]
    per_w = total // _NW
    n_chunks = per_w // chunk
    mesh = plsc.VectorSubcoreMesh(core_axis_name="c", subcore_axis_name="s")

    @functools.partial(
        pl.kernel,
        mesh=mesh,
        out_type=jax.ShapeDtypeStruct((total, d), jnp.float32),
        scratch_types=[
            pltpu.VMEM((chunk,), jnp.int32),
            pltpu.VMEM((chunk,), jnp.int32),
            pltpu.VMEM((chunk, d), jnp.float32),
            pltpu.VMEM((chunk, d), jnp.float32),
            pltpu.SemaphoreType.DMA,
            pltpu.SemaphoreType.DMA,
        ],
    )
    def k(t1_hbm, i1_hbm, t2_hbm, i2_hbm, out_hbm, i1_v, i2_v, a_v, b_v, s1, s2):
        wid = lax.axis_index("s") * _NC + lax.axis_index("c")
        base_w = wid * per_w

        def chunk_body(ci, carry):
            base = base_w + ci * chunk
            pltpu.sync_copy(i1_hbm.at[pl.ds(base, chunk)], i1_v)
            pltpu.sync_copy(i2_hbm.at[pl.ds(base, chunk)], i2_v)
            c1 = pltpu.async_copy(t1_hbm.at[i1_v], a_v, s1)
            c2 = pltpu.async_copy(t2_hbm.at[i2_v], b_v, s2)
            c1.wait()
            c2.wait()

            def row_body(r4, rcarry):
                for u in range(4):
                    r = r4 * 4 + u
                    for kk in range(d // 16):
                        sl = pl.ds(kk * 16, 16)
                        a_v[r, sl] = a_v[r, sl] + b_v[r, sl]
                return rcarry

            lax.fori_loop(0, chunk // 4, row_body, 0)
            pltpu.sync_copy(a_v, out_hbm.at[pl.ds(base, chunk)])
            return carry

        lax.fori_loop(0, n_chunks, chunk_body, 0)

    return k(table1, idx1, table2, idx2)


def kernel(x, edge_index, edge_attr, atom_emb1, atom_emb2, edge_emb1, edge_emb2, W1, b1, W2, b2, gamma, beta):
    n = x.shape[0]
    d = atom_emb1.shape[1]
    e = edge_index.shape[1]
    n_pad = ((n + 8 * _NW - 1) // (8 * _NW)) * (8 * _NW)

    pad = n_pad - n
    x0p = jnp.concatenate([x[:, 0], jnp.zeros((pad,), dtype=x.dtype)])
    x1p = jnp.concatenate([x[:, 1], jnp.zeros((pad,), dtype=x.dtype)])
    h = _gather2_add(atom_emb1, x0p, atom_emb2, x1p, n_pad // _NW)[:n]

    src = edge_index[0]
    dst = edge_index[1]
    order = jnp.argsort(dst, stable=True)
    src_s = src[order]
    dst_s = dst[order]
    n2 = edge_emb2.shape[1]
    cidx_s = (edge_attr[:, 0] * n2 + edge_attr[:, 1])[order]
    n12 = edge_emb1.shape[1] * n2
    eidx_rep = (jnp.arange(e, dtype=jnp.int32) // (e // _NW)) * n12 + cidx_s

    for l in range(_NUM_LAYERS):
        e12 = (edge_emb1[l][:, None, :] + edge_emb2[l][None, :, :]).reshape(n12, d)
        e12rep = jnp.tile(e12, (_NW, 1))
        msg = _gather2_add(h, src_s, e12rep, eidx_rep, 400)
        agg = jax.ops.segment_sum(msg, dst_s, num_segments=n)
        self_loop_emb = edge_emb1[l][4] + edge_emb2[l][0]
        agg = agg + h + self_loop_emb[None, :]
        hh = jnp.maximum(agg @ W1[l] + b1[l], 0.0) @ W2[l] + b2[l]
        mean = hh.mean(axis=0)
        var = hh.var(axis=0)
        hh = (hh - mean) / jnp.sqrt(var + 1e-5) * gamma[l] + beta[l]
        h = hh if l == _NUM_LAYERS - 1 else jnp.maximum(hh, 0.0)
    pooled = h.mean(axis=0, keepdims=True)
    return pooled
